# baseline (speedup 1.0000x reference)
import functools

import jax
import jax.numpy as jnp
import numpy as np
from jax.experimental import pallas as pl

N, E, D, H, De = 10000, 320000, 128, 256, 16


def _layer_body(S_ref, cnt_ref, h_ref, Wl_ref, Wr_ref, b_ref, o_ref, *, act):
    c = jnp.maximum(cnt_ref[...], 1.0)
    agg = S_ref[...] / c
    u = (jnp.dot(agg, Wl_ref[...], preferred_element_type=jnp.float32)
         + jnp.dot(h_ref[...], Wr_ref[...], preferred_element_type=jnp.float32)
         + b_ref[...])
    o_ref[...] = jnp.maximum(u, 0.0) if act else u


def _sage_layer(S, cnt, h, Wl, b, Wr, act):
    n, din = S.shape
    hh = Wl.shape[1]
    b = b.reshape(1, hh)
    blk = 512
    grid = (pl.cdiv(n, blk),)
    return pl.pallas_call(
        functools.partial(_layer_body, act=act),
        grid=grid,
        in_specs=[
            pl.BlockSpec((blk, din), lambda i: (i, 0)),
            pl.BlockSpec((blk, 1), lambda i: (i, 0)),
            pl.BlockSpec((blk, din), lambda i: (i, 0)),
            pl.BlockSpec((din, hh), lambda i: (0, 0)),
            pl.BlockSpec((din, hh), lambda i: (0, 0)),
            pl.BlockSpec((1, hh), lambda i: (0, 0)),
        ],
        out_specs=pl.BlockSpec((blk, hh), lambda i: (i, 0)),
        out_shape=jax.ShapeDtypeStruct((n, hh), jnp.float32),
    )(S, cnt, h, Wl, Wr, b)


def _layer2_body(S_ref, cnt_ref, h_ref, Wl_ref, Wr_ref, b_ref, Wa_ref, Wb_ref,
                 hs_ref, hd_ref):
    c = jnp.maximum(cnt_ref[...], 1.0)
    agg = S_ref[...] / c
    u = (jnp.dot(agg, Wl_ref[...], preferred_element_type=jnp.float32)
         + jnp.dot(h_ref[...], Wr_ref[...], preferred_element_type=jnp.float32)
         + b_ref[...])
    hs_ref[...] = jnp.dot(u, Wa_ref[...], preferred_element_type=jnp.float32)
    hd_ref[...] = jnp.dot(u, Wb_ref[...], preferred_element_type=jnp.float32)


def _sage_layer2(S, cnt, h, Wl, b, Wr, Wa, Wb):
    n, din = S.shape
    hh = Wl.shape[1]
    b = b.reshape(1, hh)
    blk = 512
    grid = (pl.cdiv(n, blk),)
    return pl.pallas_call(
        _layer2_body,
        grid=grid,
        in_specs=[
            pl.BlockSpec((blk, din), lambda i: (i, 0)),
            pl.BlockSpec((blk, 1), lambda i: (i, 0)),
            pl.BlockSpec((blk, din), lambda i: (i, 0)),
            pl.BlockSpec((din, hh), lambda i: (0, 0)),
            pl.BlockSpec((din, hh), lambda i: (0, 0)),
            pl.BlockSpec((1, hh), lambda i: (0, 0)),
            pl.BlockSpec((hh, hh), lambda i: (0, 0)),
            pl.BlockSpec((hh, hh), lambda i: (0, 0)),
        ],
        out_specs=[
            pl.BlockSpec((blk, hh), lambda i: (i, 0)),
            pl.BlockSpec((blk, hh), lambda i: (i, 0)),
        ],
        out_shape=[
            jax.ShapeDtypeStruct((n, hh), jnp.float32),
            jax.ShapeDtypeStruct((n, hh), jnp.float32),
        ],
    )(S, cnt, h, Wl, Wr, b, Wa, Wb)


def _head_body(g_ref, ea_ref, Wea_ref, b1_ref, W2_ref, b2_ref, W3_ref, b3_ref,
               o_ref):
    z1 = g_ref[...] + jnp.dot(ea_ref[...], Wea_ref[...],
                              preferred_element_type=jnp.float32) + b1_ref[...]
    z1 = jnp.maximum(z1, 0.0)
    z2 = jnp.dot(z1, W2_ref[...], preferred_element_type=jnp.float32) + b2_ref[...]
    z2 = jnp.maximum(z2, 0.0)
    o_ref[...] = jnp.dot(z2, W3_ref[...], preferred_element_type=jnp.float32) + b3_ref[...]


def _edge_head(g, ea, Wea, b1, W2, b2, W3, b3):
    e = g.shape[0]
    blk = 1024
    grid = (pl.cdiv(e, blk),)
    return pl.pallas_call(
        _head_body,
        grid=grid,
        in_specs=[
            pl.BlockSpec((blk, H), lambda i: (i, 0)),
            pl.BlockSpec((blk, De), lambda i: (i, 0)),
            pl.BlockSpec((De, H), lambda i: (0, 0)),
            pl.BlockSpec((1, H), lambda i: (0, 0)),
            pl.BlockSpec((H, H // 2), lambda i: (0, 0)),
            pl.BlockSpec((1, H // 2), lambda i: (0, 0)),
            pl.BlockSpec((H // 2, 128), lambda i: (0, 0)),
            pl.BlockSpec((1, 128), lambda i: (0, 0)),
        ],
        out_specs=pl.BlockSpec((blk, 128), lambda i: (i, 0)),
        out_shape=jax.ShapeDtypeStruct((e, 128), jnp.float32),
    )(g, ea, Wea, b1, W2, b2, W3, b3)


def kernel(x, edge_index, edge_attr, W0l, b0l, W0r, W1l, b1l, W1r, W2l, b2l,
           W2r, g0, be0, g1, be1, Wc1, bc1, Wc2, bc2, Wc3, bc3):
    src = edge_index[0]
    dst = edge_index[1]

    s0 = g0 / np.sqrt(1.0 + 1e-5)
    s1 = g1 / np.sqrt(1.0 + 1e-5)
    W0l_, W0r_, b0_ = W0l * s0[None, :], W0r * s0[None, :], b0l * s0 + be0
    W1l_, W1r_, b1_ = W1l * s1[None, :], W1r * s1[None, :], b1l * s1 + be1

    cnt = jax.ops.segment_sum(jnp.ones((E,), jnp.float32), dst, num_segments=N)
    cnt = cnt[:, None]

    S0 = jax.ops.segment_sum(x[src], dst, num_segments=N)
    h1 = _sage_layer(S0, cnt, x, W0l_, b0_, W0r_, act=True)
    S1 = jax.ops.segment_sum(h1[src], dst, num_segments=N)
    h2 = _sage_layer(S1, cnt, h1, W1l_, b1_, W1r_, act=True)
    S2 = jax.ops.segment_sum(h2[src], dst, num_segments=N)
    hs, hd = _sage_layer2(S2, cnt, h2, W2l, b2l, W2r,
                          Wc1[:H, :], Wc1[H:2 * H, :])

    g = hs[src] + hd[dst]

    Wea = Wc1[2 * H:, :]
    W3p = jnp.zeros((H // 2, 128), jnp.float32).at[:, :2].set(Wc3)
    b3p = jnp.zeros((128,), jnp.float32).at[:2].set(bc3)
    out = _edge_head(g, edge_attr, Wea, bc1[None, :], Wc2, bc2[None, :],
                     W3p, b3p[None, :])
    return out[:, :2]


# speedup vs baseline: 3.2596x; 3.2596x over previous
import jax
import jax.numpy as jnp
import numpy as np
from jax import lax
from jax.experimental import pallas as pl
from jax.experimental.pallas import tpu as pltpu
from jax.experimental.pallas import tpu_sc as plsc

N, E, D, H, De = 10000, 320000, 128, 256, 16
NC, NS = 2, 16
CH = 80
SEC = 25
F = 128
RPT = (N // NS) // 8 * 8
RLAST = N - (NS - 1) * RPT

_MESH = plsc.VectorSubcoreMesh(core_axis_name="c", subcore_axis_name="s",
                               num_cores=NC, num_subcores=NS)


def _sc_segsum(tA, tB, srcg, dstg, zrows):
    nchunk = E // NS // CH
    nsect = nchunk // SEC

    def body(tA_ref, tB_ref, srcg_ref, dstg_ref, z_ref,
             outA_ref, outB_ref, src_v, dst_v, rows_v, S_sp, sem):
        cid = lax.axis_index("c")
        sid = lax.axis_index("s")

        @pl.when(sid < NS - 1)
        def _():
            pltpu.sync_copy(z_ref.at[pl.ds(0, RPT)],
                            S_sp.at[pl.ds(sid * RPT, RPT)])

        @pl.when(sid == NS - 1)
        def _():
            pltpu.sync_copy(z_ref, S_sp.at[pl.ds((NS - 1) * RPT, RLAST)])

        plsc.subcore_barrier()

        def run(table_ref, out_ref):
            def sect(s, carry):
                pltpu.sync_copy(srcg_ref.at[sid, s], src_v)
                pltpu.sync_copy(dstg_ref.at[sid, s], dst_v)

                def chunk(j, c2):
                    pltpu.async_copy(table_ref.at[src_v.at[j]], rows_v,
                                     sem).wait()
                    pltpu.sync_copy(rows_v, S_sp.at[dst_v.at[j]], add=True)
                    return c2

                return lax.fori_loop(0, SEC, chunk, carry)

            lax.fori_loop(0, nsect, sect, 0)
            plsc.subcore_barrier()

            @pl.when(sid < NS - 1)
            def _():
                pltpu.sync_copy(S_sp.at[pl.ds(sid * RPT, RPT)],
                                out_ref.at[pl.ds(sid * RPT, RPT)])

            @pl.when(sid == NS - 1)
            def _():
                pltpu.sync_copy(S_sp.at[pl.ds((NS - 1) * RPT, RLAST)],
                                out_ref.at[pl.ds((NS - 1) * RPT, RLAST)])

        @pl.when(cid == 0)
        def _():
            run(tA_ref, outA_ref)

        @pl.when(cid == 1)
        def _():
            run(tB_ref, outB_ref)

    f = pl.kernel(
        body,
        out_type=[jax.ShapeDtypeStruct((N, F), jnp.float32),
                  jax.ShapeDtypeStruct((N, F), jnp.float32)],
        mesh=_MESH,
        scratch_types=[
            pltpu.VMEM((SEC, CH), jnp.int32),
            pltpu.VMEM((SEC, CH), jnp.int32),
            pltpu.VMEM((CH, F), jnp.float32),
            pltpu.VMEM_SHARED((N, F), jnp.float32),
            pltpu.SemaphoreType.DMA,
        ],
    )
    return f(tA, tB, srcg, dstg, zrows)


def _sc_edge_gather(hs, hd, srcw, dstw):
    epw = E // (NC * NS)
    nchunk = epw // CH

    def body(hs_ref, hd_ref, srcw_ref, dstw_ref, o1_ref, o2_ref,
             src_v, dst_v, rows_v, sem):
        cid = lax.axis_index("c")
        sid = lax.axis_index("s")
        wid = cid * NS + sid
        pltpu.sync_copy(srcw_ref.at[wid], src_v)
        pltpu.sync_copy(dstw_ref.at[wid], dst_v)

        def chunk(j, carry):
            pltpu.async_copy(hs_ref.at[src_v.at[j]], rows_v, sem).wait()
            pltpu.sync_copy(rows_v, o1_ref.at[pl.ds(wid * epw + j * CH, CH)])
            pltpu.async_copy(hd_ref.at[dst_v.at[j]], rows_v, sem).wait()
            pltpu.sync_copy(rows_v, o2_ref.at[pl.ds(wid * epw + j * CH, CH)])
            return carry

        lax.fori_loop(0, nchunk, chunk, 0)

    f = pl.kernel(
        body,
        out_type=[jax.ShapeDtypeStruct((E, H), jnp.float32),
                  jax.ShapeDtypeStruct((E, H), jnp.float32)],
        mesh=_MESH,
        scratch_types=[
            pltpu.VMEM((nchunk, CH), jnp.int32),
            pltpu.VMEM((nchunk, CH), jnp.int32),
            pltpu.VMEM((CH, H), jnp.float32),
            pltpu.SemaphoreType.DMA,
        ],
    )
    return f(hs, hd, srcw, dstw)


def _sage_body(nq, relu, S0, S1, *refs):
    S_refs = refs[:nq]
    cnt_ref = refs[nq]
    h_refs = refs[nq + 1:2 * nq + 1]
    Wl_refs = refs[2 * nq + 1:3 * nq + 1]
    Wr_refs = refs[3 * nq + 1:4 * nq + 1]
    b_ref = refs[4 * nq + 1]
    o_refs = refs[4 * nq + 2:]
    c = jnp.maximum(cnt_ref[...], 1.0)
    u = b_ref[...]
    for q in range(nq):
        u = u + jnp.dot(S_refs[q][...] / c, Wl_refs[q][...],
                        preferred_element_type=jnp.float32)
        u = u + jnp.dot(h_refs[q][...], Wr_refs[q][...],
                        preferred_element_type=jnp.float32)
    if relu:
        u = jnp.maximum(u, 0.0)
    if S0 is None:
        fo = H // len(o_refs)
        for q in range(len(o_refs)):
            o_refs[q][...] = u[:, fo * q:fo * (q + 1)]
    else:
        o_refs[0][...] = jnp.dot(u, S0[...], preferred_element_type=jnp.float32)
        o_refs[1][...] = jnp.dot(u, S1[...], preferred_element_type=jnp.float32)


def _sage_layer(Ss, cnt, hs, Wls, Wrs, b, relu=True, fin=None):
    nq = len(Ss)
    blk = 512
    grid = (pl.cdiv(N, blk),)
    fS = Ss[0].shape[1]
    fh = hs[0].shape[1]
    in_specs = (
        [pl.BlockSpec((blk, fS), lambda i: (i, 0)) for _ in range(nq)]
        + [pl.BlockSpec((blk, 1), lambda i: (i, 0))]
        + [pl.BlockSpec((blk, fh), lambda i: (i, 0)) for _ in range(nq)]
        + [pl.BlockSpec((fS, H), lambda i: (0, 0)) for _ in range(nq)]
        + [pl.BlockSpec((fh, H), lambda i: (0, 0)) for _ in range(nq)]
        + [pl.BlockSpec((1, H), lambda i: (0, 0))]
    )
    if fin is None:
        out_specs = [pl.BlockSpec((blk, H // 2), lambda i: (i, 0))
                     for _ in range(2)]
        out_shape = [jax.ShapeDtypeStruct((N, H // 2), jnp.float32)
                     for _ in range(2)]
        body = lambda *refs: _sage_body(nq, relu, None, None, *refs)
        extra = []
    else:
        out_specs = [pl.BlockSpec((blk, H), lambda i: (i, 0))
                     for _ in range(2)]
        out_shape = [jax.ShapeDtypeStruct((N, H), jnp.float32)
                     for _ in range(2)]
        in_specs = in_specs + [pl.BlockSpec((H, H), lambda i: (0, 0))
                               for _ in range(2)]
        body = lambda *refs: _sage_body(nq, relu, refs[4 * nq + 2],
                                        refs[4 * nq + 3], *refs[:4 * nq + 2],
                                        *refs[4 * nq + 4:])
        extra = list(fin)
    return pl.pallas_call(
        body,
        grid=grid,
        in_specs=in_specs,
        out_specs=out_specs,
        out_shape=out_shape,
    )(*Ss, cnt, *hs, *Wls, *Wrs, b, *extra)


def _head_body(g1_ref, g2_ref, ea_ref, Wea_ref, b1_ref, W2_ref, b2_ref,
               W3_ref, b3_ref, o_ref):
    z1 = (g1_ref[...] + g2_ref[...]
          + jnp.dot(ea_ref[...], Wea_ref[...],
                    preferred_element_type=jnp.float32) + b1_ref[...])
    z1 = jnp.maximum(z1, 0.0)
    z2 = jnp.dot(z1, W2_ref[...], preferred_element_type=jnp.float32) + b2_ref[...]
    z2 = jnp.maximum(z2, 0.0)
    o_ref[...] = jnp.dot(z2, W3_ref[...], preferred_element_type=jnp.float32) + b3_ref[...]


def _edge_head(g1, g2, ea, Wea, b1, W2, b2, W3, b3):
    blk = 1024
    grid = (pl.cdiv(E, blk),)
    return pl.pallas_call(
        _head_body,
        grid=grid,
        in_specs=[
            pl.BlockSpec((blk, H), lambda i: (i, 0)),
            pl.BlockSpec((blk, H), lambda i: (i, 0)),
            pl.BlockSpec((blk, De), lambda i: (i, 0)),
            pl.BlockSpec((De, H), lambda i: (0, 0)),
            pl.BlockSpec((1, H), lambda i: (0, 0)),
            pl.BlockSpec((H, H // 2), lambda i: (0, 0)),
            pl.BlockSpec((1, H // 2), lambda i: (0, 0)),
            pl.BlockSpec((H // 2, 128), lambda i: (0, 0)),
            pl.BlockSpec((1, 128), lambda i: (0, 0)),
        ],
        out_specs=pl.BlockSpec((blk, 128), lambda i: (i, 0)),
        out_shape=jax.ShapeDtypeStruct((E, 128), jnp.float32),
    )(g1, g2, ea, Wea, b1, W2, b2, W3, b3)


def kernel(x, edge_index, edge_attr, W0l, b0l, W0r, W1l, b1l, W1r, W2l, b2l,
           W2r, g0, be0, g1, be1, Wc1, bc1, Wc2, bc2, Wc3, bc3):
    src = edge_index[0]
    dst = edge_index[1]
    nsect = E // NS // CH // SEC
    srcg = src.reshape(NS, nsect, SEC, CH)
    dstg = dst.reshape(NS, nsect, SEC, CH)
    srcw = src.reshape(NC * NS, E // (NC * NS) // CH, CH)
    dstw = dst.reshape(NC * NS, E // (NC * NS) // CH, CH)

    s0 = g0 / np.sqrt(1.0 + 1e-5)
    s1 = g1 / np.sqrt(1.0 + 1e-5)
    W0l_, W0r_, b0_ = W0l * s0[None, :], W0r * s0[None, :], (b0l * s0 + be0)
    W1l_, W1r_, b1_ = W1l * s1[None, :], W1r * s1[None, :], (b1l * s1 + be1)

    onespad = jnp.concatenate(
        [jnp.ones((N, 1), jnp.float32), jnp.zeros((N, F - 65), jnp.float32)], 1)
    t0A = jnp.concatenate([x[:, :64], onespad], axis=1)
    t0B = jnp.concatenate([x[:, 64:], jnp.zeros((N, F - 64), jnp.float32)], axis=1)
    zrows = jnp.zeros((RLAST, F), jnp.float32)

    S0A, S0B = _sc_segsum(t0A, t0B, srcg, dstg, zrows)
    cnt = S0A[:, 64:65]

    WlA0 = jnp.concatenate([W0l_[:64], jnp.zeros((F - 64, H), jnp.float32)], 0)
    WlB0 = jnp.concatenate([W0l_[64:], jnp.zeros((F - 64, H), jnp.float32)], 0)
    h1 = _sage_layer([S0A, S0B], cnt, [x[:, :64], x[:, 64:]],
                     [WlA0, WlB0], [W0r_[:64], W0r_[64:]], b0_.reshape(1, H))

    S1A, S1B = _sc_segsum(h1[0], h1[1], srcg, dstg, zrows)
    W1lh = [W1l_[:F], W1l_[F:]]
    W1rh = [W1r_[:F], W1r_[F:]]
    h2 = _sage_layer([S1A, S1B], cnt, list(h1), W1lh, W1rh, b1_.reshape(1, H))

    S2A, S2B = _sc_segsum(h2[0], h2[1], srcg, dstg, zrows)
    W2lh = [W2l[:F], W2l[F:]]
    W2rh = [W2r[:F], W2r[F:]]
    hs, hd = _sage_layer([S2A, S2B], cnt, list(h2), W2lh, W2rh,
                         b2l.reshape(1, H), relu=False,
                         fin=(Wc1[:H, :], Wc1[H:2 * H, :]))

    g1e, g2e = _sc_edge_gather(hs, hd, srcw, dstw)

    Wea = Wc1[2 * H:, :]
    W3p = jnp.zeros((H // 2, 128), jnp.float32).at[:, :2].set(Wc3)
    b3p = jnp.zeros((128,), jnp.float32).at[:2].set(bc3)
    out = _edge_head(g1e, g2e, edge_attr, Wea, bc1[None, :], Wc2, bc2[None, :],
                     W3p, b3p[None, :])
    return out[:, :2]
